# initial kernel scaffold (unmeasured)
import jax
import jax.numpy as jnp
from jax import lax
from jax.experimental import pallas as pl
from jax.experimental.pallas import tpu as pltpu

N_DEV = 8
M = 4096
N = 2048
CH = M // N_DEV


def kernel(x, w_mat):
    partial = lax.dot_general(
        x,
        w_mat,
        (((1,), (0,)), ((), ())),
        precision=lax.Precision.HIGHEST,
        preferred_element_type=jnp.float32,
    )

    def body(p_ref, out_ref, comm_ref, pc_ref, rs_send, rs_recv, ag_send,
             ag_recv, local_sem):
        me = lax.axis_index("i")
        right = lax.rem(me + 1, N_DEV)
        left = lax.rem(me + N_DEV - 1, N_DEV)

        barrier = pltpu.get_barrier_semaphore()
        for nbr in (left, right):
            pl.semaphore_signal(
                barrier, inc=1, device_id=(nbr,),
                device_id_type=pl.DeviceIdType.MESH,
            )
        pl.semaphore_wait(barrier, 2)

        def load_pc(c):
            cp = pltpu.make_async_copy(
                p_ref.at[pl.ds(c * CH, CH), :], pc_ref, local_sem)
            cp.start()
            cp.wait()

        for s in range(N_DEV - 1):
            if s == 0:
                src = p_ref.at[pl.ds(me * CH, CH), :]
            else:
                src = comm_ref.at[s - 1]
            rdma = pltpu.make_async_remote_copy(
                src_ref=src,
                dst_ref=comm_ref.at[s],
                send_sem=rs_send.at[s],
                recv_sem=rs_recv.at[s],
                device_id=(right,),
                device_id_type=pl.DeviceIdType.MESH,
            )
            rdma.start()
            rdma.wait()
            c = lax.rem(me - s - 1 + N_DEV, N_DEV)
            load_pc(c)
            comm_ref[s] = comm_ref[s] + pc_ref[...]

        own = lax.rem(me + 1, N_DEV)
        cp = pltpu.make_async_copy(
            comm_ref.at[N_DEV - 2], out_ref.at[pl.ds(own * CH, CH), :],
            local_sem)
        cp.start()
        cp.wait()

        for t in range(N_DEV - 1):
            g = lax.rem(me + 1 - t + N_DEV, N_DEV)
            if t == 0:
                src = comm_ref.at[N_DEV - 2]
            else:
                src = out_ref.at[pl.ds(g * CH, CH), :]
            rdma = pltpu.make_async_remote_copy(
                src_ref=src,
                dst_ref=out_ref.at[pl.ds(g * CH, CH), :],
                send_sem=ag_send.at[t],
                recv_sem=ag_recv.at[t],
                device_id=(right,),
                device_id_type=pl.DeviceIdType.MESH,
            )
            rdma.start()
            rdma.wait()

        amax = jnp.float32(0.0)
        for c in range(N_DEV):
            load_pc(c)
            amax = jnp.maximum(amax, jnp.max(jnp.abs(pc_ref[...])))
        scale = amax / 127.0
        inv = 127.0 / amax
        for c in range(N_DEV):
            load_pc(c)
            q = jnp.clip(jnp.round(pc_ref[...] * inv), -127.0, 127.0)
            pc_ref[...] = q * scale
            cp = pltpu.make_async_copy(
                pc_ref, out_ref.at[pl.ds(c * CH, CH), :], local_sem)
            cp.start()
            cp.wait()

    return pl.pallas_call(
        body,
        out_shape=jax.ShapeDtypeStruct((M, N), jnp.float32),
        in_specs=[pl.BlockSpec(memory_space=pl.ANY)],
        out_specs=pl.BlockSpec(memory_space=pl.ANY),
        scratch_shapes=[
            pltpu.VMEM((N_DEV - 1, CH, N), jnp.float32),
            pltpu.VMEM((CH, N), jnp.float32),
            pltpu.SemaphoreType.DMA((N_DEV - 1,)),
            pltpu.SemaphoreType.DMA((N_DEV - 1,)),
            pltpu.SemaphoreType.DMA((N_DEV - 1,)),
            pltpu.SemaphoreType.DMA((N_DEV - 1,)),
            pltpu.SemaphoreType.DMA,
        ],
        compiler_params=pltpu.CompilerParams(collective_id=0),
    )(partial)


# baseline (device time: 823474 ns/iter reference)
import jax
import jax.numpy as jnp
from jax import lax
from jax.experimental import pallas as pl
from jax.experimental.pallas import tpu as pltpu

N_DEV = 8
M = 4096
N = 2048
CH = M // N_DEV


def kernel(x, w_mat, quant=True):
    partial = lax.dot_general(
        x,
        w_mat,
        (((1,), (0,)), ((), ())),
        precision=lax.Precision.HIGHEST,
        preferred_element_type=jnp.float32,
    )

    def body(p_ref, out_ref, comm_ref, pc_ref, rs_send, rs_recv, ag_send,
             ag_recv, local_sem):
        me = lax.axis_index("i")
        right = lax.rem(me + 1, N_DEV)
        left = lax.rem(me + N_DEV - 1, N_DEV)

        barrier = pltpu.get_barrier_semaphore()
        for nbr in (left, right):
            pl.semaphore_signal(
                barrier, inc=1, device_id=(nbr,),
                device_id_type=pl.DeviceIdType.MESH,
            )
        pl.semaphore_wait(barrier, 2)

        def load_pc(c):
            cp = pltpu.make_async_copy(
                p_ref.at[pl.ds(c * CH, CH), :], pc_ref, local_sem)
            cp.start()
            cp.wait()

        for s in range(N_DEV - 1):
            if s == 0:
                src = p_ref.at[pl.ds(me * CH, CH), :]
            else:
                src = comm_ref.at[s - 1]
            rdma = pltpu.make_async_remote_copy(
                src_ref=src,
                dst_ref=comm_ref.at[s],
                send_sem=rs_send.at[s],
                recv_sem=rs_recv.at[s],
                device_id=(right,),
                device_id_type=pl.DeviceIdType.MESH,
            )
            rdma.start()
            rdma.wait()
            c = lax.rem(me - s - 1 + N_DEV, N_DEV)
            load_pc(c)
            comm_ref[s] = comm_ref[s] + pc_ref[...]

        own = lax.rem(me + 1, N_DEV)
        cp = pltpu.make_async_copy(
            comm_ref.at[N_DEV - 2], out_ref.at[pl.ds(own * CH, CH), :],
            local_sem)
        cp.start()
        cp.wait()

        for t in range(N_DEV - 1):
            g = lax.rem(me + 1 - t + N_DEV, N_DEV)
            if t == 0:
                src = comm_ref.at[N_DEV - 2]
            else:
                src = out_ref.at[pl.ds(g * CH, CH), :]
            rdma = pltpu.make_async_remote_copy(
                src_ref=src,
                dst_ref=out_ref.at[pl.ds(g * CH, CH), :],
                send_sem=ag_send.at[t],
                recv_sem=ag_recv.at[t],
                device_id=(right,),
                device_id_type=pl.DeviceIdType.MESH,
            )
            rdma.start()
            rdma.wait()

        def load_out(c):
            cp = pltpu.make_async_copy(
                out_ref.at[pl.ds(c * CH, CH), :], pc_ref, local_sem)
            cp.start()
            cp.wait()

        if quant:
            amax = jnp.float32(0.0)
            for c in range(N_DEV):
                load_out(c)
                amax = jnp.maximum(amax, jnp.max(jnp.abs(pc_ref[...])))
            scale = amax / 127.0
            inv = 127.0 / amax
            for c in range(N_DEV):
                load_out(c)
                q = jnp.clip(jnp.round(pc_ref[...] * inv), -127.0, 127.0)
                pc_ref[...] = q * scale
                cp = pltpu.make_async_copy(
                    pc_ref, out_ref.at[pl.ds(c * CH, CH), :], local_sem)
                cp.start()
                cp.wait()

    return pl.pallas_call(
        body,
        out_shape=jax.ShapeDtypeStruct((M, N), jnp.float32),
        in_specs=[pl.BlockSpec(memory_space=pl.ANY)],
        out_specs=pl.BlockSpec(memory_space=pl.ANY),
        scratch_shapes=[
            pltpu.VMEM((N_DEV - 1, CH, N), jnp.float32),
            pltpu.VMEM((CH, N), jnp.float32),
            pltpu.SemaphoreType.DMA((N_DEV - 1,)),
            pltpu.SemaphoreType.DMA((N_DEV - 1,)),
            pltpu.SemaphoreType.DMA((N_DEV - 1,)),
            pltpu.SemaphoreType.DMA((N_DEV - 1,)),
            pltpu.SemaphoreType.DMA,
        ],
        compiler_params=pltpu.CompilerParams(
            collective_id=0, vmem_limit_bytes=60 * 1024 * 1024),
    )(partial)


# device time: 519762 ns/iter; 1.5843x vs baseline; 1.5843x over previous
import jax
import jax.numpy as jnp
from jax import lax
from jax.experimental import pallas as pl
from jax.experimental.pallas import tpu as pltpu

N_DEV = 8
M = 4096
N = 2048
CH = M // N_DEV


def kernel(x, w_mat, quant=True):
    partial = lax.dot_general(
        x,
        w_mat,
        (((1,), (0,)), ((), ())),
        precision=lax.Precision.HIGHEST,
        preferred_element_type=jnp.float32,
    )

    def body(p_ref, out_ref, comm_ref, pc_ref, qbuf_ref, ax_src, ax_buf,
             rs_send, rs_recv, ag_send, ag_recv, ax_send, ax_recv,
             pc_sem, st_sems):
        me = lax.axis_index("i")
        right = lax.rem(me + 1, N_DEV)
        left = lax.rem(me + N_DEV - 1, N_DEV)

        barrier = pltpu.get_barrier_semaphore()
        for nbr in (left, right):
            pl.semaphore_signal(
                barrier, inc=1, device_id=(nbr,),
                device_id_type=pl.DeviceIdType.MESH,
            )
        pl.semaphore_wait(barrier, 2)

        for s in range(N_DEV - 1):
            if s == 0:
                src = p_ref.at[pl.ds(me * CH, CH), :]
            else:
                src = comm_ref.at[s - 1]
            rdma = pltpu.make_async_remote_copy(
                src_ref=src,
                dst_ref=comm_ref.at[s],
                send_sem=rs_send.at[s],
                recv_sem=rs_recv.at[s],
                device_id=(right,),
                device_id_type=pl.DeviceIdType.MESH,
            )
            rdma.start()
            c = lax.rem(me - s - 1 + N_DEV, N_DEV)
            pc_cp = pltpu.make_async_copy(
                p_ref.at[pl.ds(c * CH, CH), :], pc_ref, pc_sem)
            pc_cp.start()
            rdma.wait()
            pc_cp.wait()
            comm_ref[s] = comm_ref[s] + pc_ref[...]

        own = lax.rem(me + 1, N_DEV)

        if not quant:
            cp = pltpu.make_async_copy(
                comm_ref.at[N_DEV - 2], out_ref.at[pl.ds(own * CH, CH), :],
                st_sems.at[2])
            cp.start()
            cp.wait()
            for t in range(N_DEV - 1):
                g = lax.rem(me + 1 - t + N_DEV, N_DEV)
                if t == 0:
                    src = comm_ref.at[N_DEV - 2]
                else:
                    src = out_ref.at[pl.ds(g * CH, CH), :]
                rdma = pltpu.make_async_remote_copy(
                    src_ref=src,
                    dst_ref=out_ref.at[pl.ds(g * CH, CH), :],
                    send_sem=ag_send.at[t],
                    recv_sem=ag_recv.at[t],
                    device_id=(right,),
                    device_id_type=pl.DeviceIdType.MESH,
                )
                rdma.start()
                rdma.wait()
            return

        my_amax = jnp.max(jnp.abs(comm_ref[N_DEV - 2]))
        ax_src[0, :] = jnp.full((128,), my_amax, jnp.float32)
        ax_buf[pl.ds(me, 1), :] = jnp.full((1, 128), my_amax, jnp.float32)
        ax_rdmas = []
        for k in range(1, N_DEV):
            peer = lax.rem(me + k, N_DEV)
            r = pltpu.make_async_remote_copy(
                src_ref=ax_src,
                dst_ref=ax_buf.at[pl.ds(me, 1), :],
                send_sem=ax_send.at[k - 1],
                recv_sem=ax_recv.at[k - 1],
                device_id=(peer,),
                device_id_type=pl.DeviceIdType.MESH,
            )
            r.start()
            ax_rdmas.append(r)
        for r in ax_rdmas:
            r.wait()
        amax = jnp.max(ax_buf[...])
        scale = amax / 127.0
        inv = 127.0 / amax

        qbuf_ref[own] = jnp.clip(
            jnp.round(comm_ref[N_DEV - 2] * inv), -127.0, 127.0
        ).astype(jnp.int8)
        comm_ref[2] = qbuf_ref[own].astype(jnp.float32) * scale
        cp = pltpu.make_async_copy(
            comm_ref.at[2], out_ref.at[pl.ds(own * CH, CH), :], st_sems.at[2])
        cp.start()

        def ag_rdma(t):
            g = lax.rem(me + 1 - t + N_DEV, N_DEV)
            return pltpu.make_async_remote_copy(
                src_ref=qbuf_ref.at[g],
                dst_ref=qbuf_ref.at[g],
                send_sem=ag_send.at[t],
                recv_sem=ag_recv.at[t],
                device_id=(right,),
                device_id_type=pl.DeviceIdType.MESH,
            )

        rdma = ag_rdma(0)
        rdma.start()
        for t in range(N_DEV - 1):
            rdma.wait()
            if t < N_DEV - 2:
                rdma = ag_rdma(t + 1)
                rdma.start()
            cr = lax.rem(me - t + N_DEV, N_DEV)
            slot = t % 2
            if t >= 2:
                pltpu.make_async_copy(
                    comm_ref.at[slot], out_ref.at[pl.ds(0, CH), :],
                    st_sems.at[slot]).wait()
            comm_ref[slot] = qbuf_ref[cr].astype(jnp.float32) * scale
            cp = pltpu.make_async_copy(
                comm_ref.at[slot], out_ref.at[pl.ds(cr * CH, CH), :],
                st_sems.at[slot])
            cp.start()
        for sl in range(3):
            pltpu.make_async_copy(
                comm_ref.at[min(sl, 2)], out_ref.at[pl.ds(0, CH), :],
                st_sems.at[sl]).wait()

    return pl.pallas_call(
        body,
        out_shape=jax.ShapeDtypeStruct((M, N), jnp.float32),
        in_specs=[pl.BlockSpec(memory_space=pl.ANY)],
        out_specs=pl.BlockSpec(memory_space=pl.ANY),
        scratch_shapes=[
            pltpu.VMEM((N_DEV - 1, CH, N), jnp.float32),
            pltpu.VMEM((CH, N), jnp.float32),
            pltpu.VMEM((N_DEV, CH, N), jnp.int8),
            pltpu.VMEM((1, 128), jnp.float32),
            pltpu.VMEM((N_DEV, 128), jnp.float32),
            pltpu.SemaphoreType.DMA((N_DEV - 1,)),
            pltpu.SemaphoreType.DMA((N_DEV - 1,)),
            pltpu.SemaphoreType.DMA((N_DEV - 1,)),
            pltpu.SemaphoreType.DMA((N_DEV - 1,)),
            pltpu.SemaphoreType.DMA((N_DEV - 1,)),
            pltpu.SemaphoreType.DMA((N_DEV - 1,)),
            pltpu.SemaphoreType.DMA,
            pltpu.SemaphoreType.DMA((3,)),
        ],
        compiler_params=pltpu.CompilerParams(
            collective_id=0, vmem_limit_bytes=60 * 1024 * 1024),
    )(partial)


# device time: 327977 ns/iter; 2.5108x vs baseline; 1.5848x over previous
import jax
import jax.numpy as jnp
from jax import lax
from jax.experimental import pallas as pl
from jax.experimental.pallas import tpu as pltpu

N_DEV = 8
M = 4096
N = 2048
HN = N // 2
CH = M // N_DEV


def mod(v):
    return lax.rem(v + 2 * N_DEV, N_DEV)


def kernel(x, w_mat, quant=True):
    del quant
    partial = lax.dot_general(
        x,
        w_mat,
        (((1,), (0,)), ((), ())),
        precision=lax.Precision.HIGHEST,
        preferred_element_type=jnp.float32,
    )

    def body(p_ref, out_ref, cA, cB, pcA, pcB, qA, qB, ax_src, ax_buf,
             rsA_s, rsA_r, rsB_s, rsB_r, agA_s, agA_r, agB_s, agB_r,
             ax_s, ax_r, pcsA, pcsB, stA, stB):
        me = lax.axis_index("i")
        right = mod(me + 1)
        left = mod(me - 1)

        barrier = pltpu.get_barrier_semaphore()
        for nbr in (left, right):
            pl.semaphore_signal(
                barrier, inc=1, device_id=(nbr,),
                device_id_type=pl.DeviceIdType.MESH,
            )
        pl.semaphore_wait(barrier, 2)

        for s in range(N_DEV - 1):
            if s == 0:
                srcA = p_ref.at[pl.ds(me * CH, CH), pl.ds(0, HN)]
                srcB = p_ref.at[pl.ds(me * CH, CH), pl.ds(HN, HN)]
            else:
                srcA = cA.at[s - 1]
                srcB = cB.at[s - 1]
            rdmaA = pltpu.make_async_remote_copy(
                src_ref=srcA, dst_ref=cA.at[s],
                send_sem=rsA_s.at[s], recv_sem=rsA_r.at[s],
                device_id=(right,), device_id_type=pl.DeviceIdType.MESH)
            rdmaB = pltpu.make_async_remote_copy(
                src_ref=srcB, dst_ref=cB.at[s],
                send_sem=rsB_s.at[s], recv_sem=rsB_r.at[s],
                device_id=(left,), device_id_type=pl.DeviceIdType.MESH)
            rdmaA.start()
            rdmaB.start()
            a_c = mod(me - s - 1)
            b_c = mod(me + s + 1)
            pcA_cp = pltpu.make_async_copy(
                p_ref.at[pl.ds(a_c * CH, CH), pl.ds(0, HN)], pcA, pcsA)
            pcB_cp = pltpu.make_async_copy(
                p_ref.at[pl.ds(b_c * CH, CH), pl.ds(HN, HN)], pcB, pcsB)
            pcA_cp.start()
            pcB_cp.start()
            rdmaA.wait()
            rdmaB.wait()
            pcA_cp.wait()
            pcB_cp.wait()
            cA[s] = cA[s] + pcA[...]
            cB[s] = cB[s] + pcB[...]

        ownA = mod(me + 1)
        ownB = mod(me - 1)

        my_amax = jnp.maximum(
            jnp.max(jnp.abs(cA[N_DEV - 2])), jnp.max(jnp.abs(cB[N_DEV - 2])))
        ax_src[0, :] = jnp.full((128,), my_amax, jnp.float32)
        ax_buf[pl.ds(me, 1), :] = jnp.full((1, 128), my_amax, jnp.float32)
        ax_rdmas = []
        for k in range(1, N_DEV):
            peer = mod(me + k)
            r = pltpu.make_async_remote_copy(
                src_ref=ax_src, dst_ref=ax_buf.at[pl.ds(me, 1), :],
                send_sem=ax_s.at[k - 1], recv_sem=ax_r.at[k - 1],
                device_id=(peer,), device_id_type=pl.DeviceIdType.MESH)
            r.start()
            ax_rdmas.append(r)
        for r in ax_rdmas:
            r.wait()
        amax = jnp.max(ax_buf[...])
        scale = amax / 127.0
        inv = 127.0 / amax

        qA[ownA] = jnp.clip(
            jnp.round(cA[N_DEV - 2] * inv), -127.0, 127.0).astype(jnp.int8)
        qB[ownB] = jnp.clip(
            jnp.round(cB[N_DEV - 2] * inv), -127.0, 127.0).astype(jnp.int8)
        cA[2] = qA[ownA].astype(jnp.float32) * scale
        cB[2] = qB[ownB].astype(jnp.float32) * scale
        pltpu.make_async_copy(
            cA.at[2], out_ref.at[pl.ds(ownA * CH, CH), pl.ds(0, HN)],
            stA.at[2]).start()
        pltpu.make_async_copy(
            cB.at[2], out_ref.at[pl.ds(ownB * CH, CH), pl.ds(HN, HN)],
            stB.at[2]).start()

        def rdA(t):
            g = mod(me + 1 - t)
            return pltpu.make_async_remote_copy(
                src_ref=qA.at[g], dst_ref=qA.at[g],
                send_sem=agA_s.at[t], recv_sem=agA_r.at[t],
                device_id=(right,), device_id_type=pl.DeviceIdType.MESH)

        def rdB(t):
            g = mod(me - 1 + t)
            return pltpu.make_async_remote_copy(
                src_ref=qB.at[g], dst_ref=qB.at[g],
                send_sem=agB_s.at[t], recv_sem=agB_r.at[t],
                device_id=(left,), device_id_type=pl.DeviceIdType.MESH)

        ra = rdA(0)
        rb = rdB(0)
        ra.start()
        rb.start()
        for t in range(N_DEV - 1):
            ra.wait()
            rb.wait()
            if t < N_DEV - 2:
                ra = rdA(t + 1)
                rb = rdB(t + 1)
                ra.start()
                rb.start()
            crA = mod(me - t)
            crB = mod(me + t)
            slot = t % 2
            if t >= 2:
                pltpu.make_async_copy(
                    cA.at[slot], out_ref.at[pl.ds(0, CH), pl.ds(0, HN)],
                    stA.at[slot]).wait()
                pltpu.make_async_copy(
                    cB.at[slot], out_ref.at[pl.ds(0, CH), pl.ds(HN, HN)],
                    stB.at[slot]).wait()
            cA[slot] = qA[crA].astype(jnp.float32) * scale
            cB[slot] = qB[crB].astype(jnp.float32) * scale
            pltpu.make_async_copy(
                cA.at[slot], out_ref.at[pl.ds(crA * CH, CH), pl.ds(0, HN)],
                stA.at[slot]).start()
            pltpu.make_async_copy(
                cB.at[slot], out_ref.at[pl.ds(crB * CH, CH), pl.ds(HN, HN)],
                stB.at[slot]).start()
        for sl in range(3):
            pltpu.make_async_copy(
                cA.at[min(sl, 2)], out_ref.at[pl.ds(0, CH), pl.ds(0, HN)],
                stA.at[sl]).wait()
            pltpu.make_async_copy(
                cB.at[min(sl, 2)], out_ref.at[pl.ds(0, CH), pl.ds(HN, HN)],
                stB.at[sl]).wait()

    nsl = N_DEV - 1
    return pl.pallas_call(
        body,
        out_shape=jax.ShapeDtypeStruct((M, N), jnp.float32),
        in_specs=[pl.BlockSpec(memory_space=pl.ANY)],
        out_specs=pl.BlockSpec(memory_space=pl.ANY),
        scratch_shapes=[
            pltpu.VMEM((nsl, CH, HN), jnp.float32),
            pltpu.VMEM((nsl, CH, HN), jnp.float32),
            pltpu.VMEM((CH, HN), jnp.float32),
            pltpu.VMEM((CH, HN), jnp.float32),
            pltpu.VMEM((N_DEV, CH, HN), jnp.int8),
            pltpu.VMEM((N_DEV, CH, HN), jnp.int8),
            pltpu.VMEM((1, 128), jnp.float32),
            pltpu.VMEM((N_DEV, 128), jnp.float32),
            pltpu.SemaphoreType.DMA((nsl,)),
            pltpu.SemaphoreType.DMA((nsl,)),
            pltpu.SemaphoreType.DMA((nsl,)),
            pltpu.SemaphoreType.DMA((nsl,)),
            pltpu.SemaphoreType.DMA((nsl,)),
            pltpu.SemaphoreType.DMA((nsl,)),
            pltpu.SemaphoreType.DMA((nsl,)),
            pltpu.SemaphoreType.DMA((nsl,)),
            pltpu.SemaphoreType.DMA((nsl,)),
            pltpu.SemaphoreType.DMA((nsl,)),
            pltpu.SemaphoreType.DMA,
            pltpu.SemaphoreType.DMA,
            pltpu.SemaphoreType.DMA((3,)),
            pltpu.SemaphoreType.DMA((3,)),
        ],
        compiler_params=pltpu.CompilerParams(
            collective_id=0, vmem_limit_bytes=60 * 1024 * 1024),
    )(partial)


# device time: 270227 ns/iter; 3.0473x vs baseline; 1.2137x over previous
import jax
import jax.numpy as jnp
from jax import lax
from jax.experimental import pallas as pl
from jax.experimental.pallas import tpu as pltpu

N_DEV = 8
M = 4096
K = 512
N = 2048
HN = N // 2
CH = M // N_DEV


def mod(v):
    return lax.rem(v + 2 * N_DEV, N_DEV)


def perm(v):
    return jnp.where(v < 4, v, 11 - v)


def kernel(x, w_mat, quant=True):
    del quant

    def body(x_ref, w_ref, out_ref, cA, cB, pcA, pcB, g0A, g0B, qA, qB,
             ax_src, ax_buf, rsA_s, rsA_r, rsB_s, rsB_r, agA_s, agA_r,
             agB_s, agB_r, ax_s, ax_r, stA, stB):
        me = lax.axis_index("i")
        p = perm(me)
        right = perm(mod(p + 1))
        left = perm(mod(p - 1))

        barrier = pltpu.get_barrier_semaphore()
        for nbr in (left, right):
            pl.semaphore_signal(
                barrier, inc=1, device_id=(nbr,),
                device_id_type=pl.DeviceIdType.MESH,
            )
        pl.semaphore_wait(barrier, 2)

        def pchunk(c, col0):
            return lax.dot_general(
                x_ref[pl.ds(c * CH, CH), :],
                w_ref[:, pl.ds(col0, HN)],
                (((1,), (0,)), ((), ())),
                precision=lax.Precision.HIGHEST,
                preferred_element_type=jnp.float32,
            )

        g0A[...] = pchunk(p, 0)
        g0B[...] = pchunk(p, HN)
        for s in range(N_DEV - 1):
            srcA = g0A if s == 0 else cA.at[s - 1]
            srcB = g0B if s == 0 else cB.at[s - 1]
            rdmaA = pltpu.make_async_remote_copy(
                src_ref=srcA, dst_ref=cA.at[s],
                send_sem=rsA_s.at[s], recv_sem=rsA_r.at[s],
                device_id=(right,), device_id_type=pl.DeviceIdType.MESH)
            rdmaB = pltpu.make_async_remote_copy(
                src_ref=srcB, dst_ref=cB.at[s],
                send_sem=rsB_s.at[s], recv_sem=rsB_r.at[s],
                device_id=(left,), device_id_type=pl.DeviceIdType.MESH)
            rdmaA.start()
            rdmaB.start()
            pcA[...] = pchunk(mod(p - s - 1), 0)
            pcB[...] = pchunk(mod(p + s + 1), HN)
            rdmaA.wait()
            rdmaB.wait()
            cA[s] = cA[s] + pcA[...]
            cB[s] = cB[s] + pcB[...]

        ownA = mod(p + 1)
        ownB = mod(p - 1)

        my_amax = jnp.maximum(
            jnp.max(jnp.abs(cA[N_DEV - 2])), jnp.max(jnp.abs(cB[N_DEV - 2])))
        ax_src[0, :] = jnp.full((128,), my_amax, jnp.float32)
        ax_buf[pl.ds(me, 1), :] = jnp.full((1, 128), my_amax, jnp.float32)
        ax_rdmas = []
        for k in range(1, N_DEV):
            peer = mod(me + k)
            r = pltpu.make_async_remote_copy(
                src_ref=ax_src, dst_ref=ax_buf.at[pl.ds(me, 1), :],
                send_sem=ax_s.at[k - 1], recv_sem=ax_r.at[k - 1],
                device_id=(peer,), device_id_type=pl.DeviceIdType.MESH)
            r.start()
            ax_rdmas.append(r)
        for r in ax_rdmas:
            r.wait()
        amax = jnp.max(ax_buf[...])
        scale = amax / 127.0
        inv = 127.0 / amax

        qA[ownA] = jnp.clip(
            jnp.round(cA[N_DEV - 2] * inv), -127.0, 127.0).astype(jnp.int8)
        qB[ownB] = jnp.clip(
            jnp.round(cB[N_DEV - 2] * inv), -127.0, 127.0).astype(jnp.int8)
        cA[2] = qA[ownA].astype(jnp.float32) * scale
        cB[2] = qB[ownB].astype(jnp.float32) * scale
        pltpu.make_async_copy(
            cA.at[2], out_ref.at[pl.ds(ownA * CH, CH), pl.ds(0, HN)],
            stA.at[2]).start()
        pltpu.make_async_copy(
            cB.at[2], out_ref.at[pl.ds(ownB * CH, CH), pl.ds(HN, HN)],
            stB.at[2]).start()

        def rdA(t):
            g = mod(p + 1 - t)
            return pltpu.make_async_remote_copy(
                src_ref=qA.at[g], dst_ref=qA.at[g],
                send_sem=agA_s.at[t], recv_sem=agA_r.at[t],
                device_id=(right,), device_id_type=pl.DeviceIdType.MESH)

        def rdB(t):
            g = mod(p - 1 + t)
            return pltpu.make_async_remote_copy(
                src_ref=qB.at[g], dst_ref=qB.at[g],
                send_sem=agB_s.at[t], recv_sem=agB_r.at[t],
                device_id=(left,), device_id_type=pl.DeviceIdType.MESH)

        ra = rdA(0)
        rb = rdB(0)
        ra.start()
        rb.start()
        for t in range(N_DEV - 1):
            ra.wait()
            rb.wait()
            if t < N_DEV - 2:
                ra = rdA(t + 1)
                rb = rdB(t + 1)
                ra.start()
                rb.start()
            crA = mod(p - t)
            crB = mod(p + t)
            slot = t % 2
            if t >= 2:
                pltpu.make_async_copy(
                    cA.at[slot], out_ref.at[pl.ds(0, CH), pl.ds(0, HN)],
                    stA.at[slot]).wait()
                pltpu.make_async_copy(
                    cB.at[slot], out_ref.at[pl.ds(0, CH), pl.ds(HN, HN)],
                    stB.at[slot]).wait()
            cA[slot] = qA[crA].astype(jnp.float32) * scale
            cB[slot] = qB[crB].astype(jnp.float32) * scale
            pltpu.make_async_copy(
                cA.at[slot], out_ref.at[pl.ds(crA * CH, CH), pl.ds(0, HN)],
                stA.at[slot]).start()
            pltpu.make_async_copy(
                cB.at[slot], out_ref.at[pl.ds(crB * CH, CH), pl.ds(HN, HN)],
                stB.at[slot]).start()
        for sl in range(3):
            pltpu.make_async_copy(
                cA.at[min(sl, 2)], out_ref.at[pl.ds(0, CH), pl.ds(0, HN)],
                stA.at[sl]).wait()
            pltpu.make_async_copy(
                cB.at[min(sl, 2)], out_ref.at[pl.ds(0, CH), pl.ds(HN, HN)],
                stB.at[sl]).wait()

    nsl = N_DEV - 1
    return pl.pallas_call(
        body,
        out_shape=jax.ShapeDtypeStruct((M, N), jnp.float32),
        in_specs=[
            pl.BlockSpec(memory_space=pltpu.VMEM),
            pl.BlockSpec(memory_space=pltpu.VMEM),
        ],
        out_specs=pl.BlockSpec(memory_space=pl.ANY),
        scratch_shapes=[
            pltpu.VMEM((nsl, CH, HN), jnp.float32),
            pltpu.VMEM((nsl, CH, HN), jnp.float32),
            pltpu.VMEM((CH, HN), jnp.float32),
            pltpu.VMEM((CH, HN), jnp.float32),
            pltpu.VMEM((CH, HN), jnp.float32),
            pltpu.VMEM((CH, HN), jnp.float32),
            pltpu.VMEM((N_DEV, CH, HN), jnp.int8),
            pltpu.VMEM((N_DEV, CH, HN), jnp.int8),
            pltpu.VMEM((1, 128), jnp.float32),
            pltpu.VMEM((N_DEV, 128), jnp.float32),
            pltpu.SemaphoreType.DMA((nsl,)),
            pltpu.SemaphoreType.DMA((nsl,)),
            pltpu.SemaphoreType.DMA((nsl,)),
            pltpu.SemaphoreType.DMA((nsl,)),
            pltpu.SemaphoreType.DMA((nsl,)),
            pltpu.SemaphoreType.DMA((nsl,)),
            pltpu.SemaphoreType.DMA((nsl,)),
            pltpu.SemaphoreType.DMA((nsl,)),
            pltpu.SemaphoreType.DMA((nsl,)),
            pltpu.SemaphoreType.DMA((nsl,)),
            pltpu.SemaphoreType.DMA((3,)),
            pltpu.SemaphoreType.DMA((3,)),
        ],
        compiler_params=pltpu.CompilerParams(
            collective_id=0, vmem_limit_bytes=60 * 1024 * 1024),
    )(x, w_mat)


# device time: 200462 ns/iter; 4.1079x vs baseline; 1.3480x over previous
import jax
import jax.numpy as jnp
from jax import lax
from jax.experimental import pallas as pl
from jax.experimental.pallas import tpu as pltpu

N_DEV = 8
M = 4096
K = 512
N = 2048
HN = N // 2
CH = M // N_DEV


def mod(v):
    return lax.rem(v + 2 * N_DEV, N_DEV)


def perm(v):
    return jnp.where(v < 4, v, 11 - v)


def kernel(x, w_mat, quant=True):
    del quant

    def body(x_ref, w_ref, out_ref, cA, cB, pcA, pcB, accA, accB, qsA, qsB,
             scsA, scsB, scbA, scbB, dqA, dqB, qA, qB, ax_src, ax_buf,
             rsA_s, rsA_r, rsB_s, rsB_r, scA_s, scA_r, scB_s, scB_r,
             agA_s, agA_r, agB_s, agB_r, ax_s, ax_r, stA, stB):
        me = lax.axis_index("i")
        p = perm(me)
        right = perm(mod(p + 1))
        left = perm(mod(p - 1))

        barrier = pltpu.get_barrier_semaphore()
        for nbr in (left, right):
            pl.semaphore_signal(
                barrier, inc=1, device_id=(nbr,),
                device_id_type=pl.DeviceIdType.MESH,
            )
        pl.semaphore_wait(barrier, 2)

        def pchunk(c, col0):
            return lax.dot_general(
                x_ref[pl.ds(c * CH, CH), :],
                w_ref[:, pl.ds(col0, HN)],
                (((1,), (0,)), ((), ())),
                precision=lax.Precision.HIGHEST,
                preferred_element_type=jnp.float32,
            )

        def q16(acc, qs_ref, scs_ref, slot):
            am = jnp.maximum(jnp.max(jnp.abs(acc)), 1e-30)
            qs_ref[slot] = jnp.clip(
                jnp.round(acc * (32767.0 / am)), -32767.0, 32767.0
            ).astype(jnp.int16)
            scs_ref[slot, :] = jnp.full((128,), am / 32767.0, jnp.float32)

        accA[...] = pchunk(p, 0)
        accB[...] = pchunk(p, HN)
        q16(accA[...], qsA, scsA, 0)
        q16(accB[...], qsB, scsB, 0)
        for s in range(N_DEV - 1):
            slot = s % 2
            rdmaA = pltpu.make_async_remote_copy(
                src_ref=qsA.at[slot], dst_ref=cA.at[s],
                send_sem=rsA_s.at[s], recv_sem=rsA_r.at[s],
                device_id=(right,), device_id_type=pl.DeviceIdType.MESH)
            rdmaB = pltpu.make_async_remote_copy(
                src_ref=qsB.at[slot], dst_ref=cB.at[s],
                send_sem=rsB_s.at[s], recv_sem=rsB_r.at[s],
                device_id=(left,), device_id_type=pl.DeviceIdType.MESH)
            scdA = pltpu.make_async_remote_copy(
                src_ref=scsA.at[pl.ds(slot, 1), :],
                dst_ref=scbA.at[pl.ds(s, 1), :],
                send_sem=scA_s.at[s], recv_sem=scA_r.at[s],
                device_id=(right,), device_id_type=pl.DeviceIdType.MESH)
            scdB = pltpu.make_async_remote_copy(
                src_ref=scsB.at[pl.ds(slot, 1), :],
                dst_ref=scbB.at[pl.ds(s, 1), :],
                send_sem=scB_s.at[s], recv_sem=scB_r.at[s],
                device_id=(left,), device_id_type=pl.DeviceIdType.MESH)
            rdmaA.start()
            rdmaB.start()
            scdA.start()
            scdB.start()
            pcA[...] = pchunk(mod(p - s - 1), 0)
            pcB[...] = pchunk(mod(p + s + 1), HN)
            rdmaA.wait()
            rdmaB.wait()
            scdA.wait()
            scdB.wait()
            accA[...] = cA[s].astype(jnp.float32) * scbA[s, 0] + pcA[...]
            accB[...] = cB[s].astype(jnp.float32) * scbB[s, 0] + pcB[...]
            if s < N_DEV - 2:
                q16(accA[...], qsA, scsA, (s + 1) % 2)
                q16(accB[...], qsB, scsB, (s + 1) % 2)

        ownA = mod(p + 1)
        ownB = mod(p - 1)

        my_amax = jnp.maximum(
            jnp.max(jnp.abs(accA[...])), jnp.max(jnp.abs(accB[...])))
        ax_src[0, :] = jnp.full((128,), my_amax, jnp.float32)
        ax_buf[pl.ds(me, 1), :] = jnp.full((1, 128), my_amax, jnp.float32)
        ax_rdmas = []
        for k in range(1, N_DEV):
            peer = mod(me + k)
            r = pltpu.make_async_remote_copy(
                src_ref=ax_src, dst_ref=ax_buf.at[pl.ds(me, 1), :],
                send_sem=ax_s.at[k - 1], recv_sem=ax_r.at[k - 1],
                device_id=(peer,), device_id_type=pl.DeviceIdType.MESH)
            r.start()
            ax_rdmas.append(r)
        for r in ax_rdmas:
            r.wait()
        amax = jnp.max(ax_buf[...])
        scale = amax / 127.0
        inv = 127.0 / amax

        qA[ownA] = jnp.clip(
            jnp.round(accA[...] * inv), -127.0, 127.0).astype(jnp.int8)
        qB[ownB] = jnp.clip(
            jnp.round(accB[...] * inv), -127.0, 127.0).astype(jnp.int8)
        pcA[...] = qA[ownA].astype(jnp.float32) * scale
        pcB[...] = qB[ownB].astype(jnp.float32) * scale
        pltpu.make_async_copy(
            pcA, out_ref.at[pl.ds(ownA * CH, CH), pl.ds(0, HN)],
            stA.at[2]).start()
        pltpu.make_async_copy(
            pcB, out_ref.at[pl.ds(ownB * CH, CH), pl.ds(HN, HN)],
            stB.at[2]).start()

        def rdA(t):
            g = mod(p + 1 - t)
            return pltpu.make_async_remote_copy(
                src_ref=qA.at[g], dst_ref=qA.at[g],
                send_sem=agA_s.at[t], recv_sem=agA_r.at[t],
                device_id=(right,), device_id_type=pl.DeviceIdType.MESH)

        def rdB(t):
            g = mod(p - 1 + t)
            return pltpu.make_async_remote_copy(
                src_ref=qB.at[g], dst_ref=qB.at[g],
                send_sem=agB_s.at[t], recv_sem=agB_r.at[t],
                device_id=(left,), device_id_type=pl.DeviceIdType.MESH)

        ra = rdA(0)
        rb = rdB(0)
        ra.start()
        rb.start()
        for t in range(N_DEV - 1):
            ra.wait()
            rb.wait()
            if t < N_DEV - 2:
                ra = rdA(t + 1)
                rb = rdB(t + 1)
                ra.start()
                rb.start()
            crA = mod(p - t)
            crB = mod(p + t)
            slot = t % 2
            if t >= 2:
                pltpu.make_async_copy(
                    dqA.at[slot], out_ref.at[pl.ds(0, CH), pl.ds(0, HN)],
                    stA.at[slot]).wait()
                pltpu.make_async_copy(
                    dqB.at[slot], out_ref.at[pl.ds(0, CH), pl.ds(HN, HN)],
                    stB.at[slot]).wait()
            dqA[slot] = qA[crA].astype(jnp.float32) * scale
            dqB[slot] = qB[crB].astype(jnp.float32) * scale
            pltpu.make_async_copy(
                dqA.at[slot], out_ref.at[pl.ds(crA * CH, CH), pl.ds(0, HN)],
                stA.at[slot]).start()
            pltpu.make_async_copy(
                dqB.at[slot], out_ref.at[pl.ds(crB * CH, CH), pl.ds(HN, HN)],
                stB.at[slot]).start()
        for sl in range(3):
            pltpu.make_async_copy(
                dqA.at[min(sl, 1)], out_ref.at[pl.ds(0, CH), pl.ds(0, HN)],
                stA.at[sl]).wait()
            pltpu.make_async_copy(
                dqB.at[min(sl, 1)], out_ref.at[pl.ds(0, CH), pl.ds(HN, HN)],
                stB.at[sl]).wait()

    nsl = N_DEV - 1
    return pl.pallas_call(
        body,
        out_shape=jax.ShapeDtypeStruct((M, N), jnp.float32),
        in_specs=[
            pl.BlockSpec(memory_space=pltpu.VMEM),
            pl.BlockSpec(memory_space=pltpu.VMEM),
        ],
        out_specs=pl.BlockSpec(memory_space=pl.ANY),
        scratch_shapes=[
            pltpu.VMEM((nsl, CH, HN), jnp.int16),
            pltpu.VMEM((nsl, CH, HN), jnp.int16),
            pltpu.VMEM((CH, HN), jnp.float32),
            pltpu.VMEM((CH, HN), jnp.float32),
            pltpu.VMEM((CH, HN), jnp.float32),
            pltpu.VMEM((CH, HN), jnp.float32),
            pltpu.VMEM((2, CH, HN), jnp.int16),
            pltpu.VMEM((2, CH, HN), jnp.int16),
            pltpu.VMEM((2, 128), jnp.float32),
            pltpu.VMEM((2, 128), jnp.float32),
            pltpu.VMEM((nsl, 128), jnp.float32),
            pltpu.VMEM((nsl, 128), jnp.float32),
            pltpu.VMEM((2, CH, HN), jnp.float32),
            pltpu.VMEM((2, CH, HN), jnp.float32),
            pltpu.VMEM((N_DEV, CH, HN), jnp.int8),
            pltpu.VMEM((N_DEV, CH, HN), jnp.int8),
            pltpu.VMEM((1, 128), jnp.float32),
            pltpu.VMEM((N_DEV, 128), jnp.float32),
            pltpu.SemaphoreType.DMA((nsl,)),
            pltpu.SemaphoreType.DMA((nsl,)),
            pltpu.SemaphoreType.DMA((nsl,)),
            pltpu.SemaphoreType.DMA((nsl,)),
            pltpu.SemaphoreType.DMA((nsl,)),
            pltpu.SemaphoreType.DMA((nsl,)),
            pltpu.SemaphoreType.DMA((nsl,)),
            pltpu.SemaphoreType.DMA((nsl,)),
            pltpu.SemaphoreType.DMA((nsl,)),
            pltpu.SemaphoreType.DMA((nsl,)),
            pltpu.SemaphoreType.DMA((nsl,)),
            pltpu.SemaphoreType.DMA((nsl,)),
            pltpu.SemaphoreType.DMA((nsl,)),
            pltpu.SemaphoreType.DMA((nsl,)),
            pltpu.SemaphoreType.DMA((3,)),
            pltpu.SemaphoreType.DMA((3,)),
        ],
        compiler_params=pltpu.CompilerParams(
            collective_id=0, vmem_limit_bytes=60 * 1024 * 1024),
    )(x, w_mat)


# device time: 197301 ns/iter; 4.1737x vs baseline; 1.0160x over previous
import jax
import jax.numpy as jnp
from jax import lax
from jax.experimental import pallas as pl
from jax.experimental.pallas import tpu as pltpu

N_DEV = 8
M = 4096
K = 512
N = 2048
HN = N // 2
CH = M // N_DEV


def mod(v):
    return lax.rem(v + 2 * N_DEV, N_DEV)


def perm(v):
    return jnp.where(v < 4, v, 11 - v)


def kernel(x, w_mat, quant=True):
    del quant

    def body(x_ref, w_ref, out_ref, cA, cB, pcA, pcB, accA, accB, qsA, qsB,
             scsA, scsB, scbA, scbB, dqA, dqB, qA, qB, ax_src, ax_buf,
             whi, wlo,
             rsA_s, rsA_r, rsB_s, rsB_r, scA_s, scA_r, scB_s, scB_r,
             agA_s, agA_r, agB_s, agB_r, ax_s, ax_r, stA, stB):
        me = lax.axis_index("i")
        p = perm(me)
        right = perm(mod(p + 1))
        left = perm(mod(p - 1))

        barrier = pltpu.get_barrier_semaphore()
        for nbr in (left, right):
            pl.semaphore_signal(
                barrier, inc=1, device_id=(nbr,),
                device_id_type=pl.DeviceIdType.MESH,
            )
        pl.semaphore_wait(barrier, 2)

        wf = w_ref[...]
        whi[...] = wf.astype(jnp.bfloat16)
        wlo[...] = (wf - whi[...].astype(jnp.float32)).astype(jnp.bfloat16)

        def bdot(a, b):
            return lax.dot_general(
                a, b, (((1,), (0,)), ((), ())),
                preferred_element_type=jnp.float32)

        def pchunk(c, col0):
            xf = x_ref[pl.ds(c * CH, CH), :]
            xhi = xf.astype(jnp.bfloat16)
            xlo = (xf - xhi.astype(jnp.float32)).astype(jnp.bfloat16)
            wh = whi[:, pl.ds(col0, HN)]
            return (bdot(xhi, wh)
                    + bdot(xhi, wlo[:, pl.ds(col0, HN)])
                    + bdot(xlo, wh))

        def q16(acc, qs_ref, scs_ref, slot):
            am = jnp.maximum(jnp.max(jnp.abs(acc)), 1e-30)
            qs_ref[slot] = jnp.clip(
                jnp.round(acc * (32767.0 / am)), -32767.0, 32767.0
            ).astype(jnp.int16)
            scs_ref[slot, :] = jnp.full((128,), am / 32767.0, jnp.float32)

        accA[...] = pchunk(p, 0)
        accB[...] = pchunk(p, HN)
        q16(accA[...], qsA, scsA, 0)
        q16(accB[...], qsB, scsB, 0)
        for s in range(N_DEV - 1):
            slot = s % 2
            rdmaA = pltpu.make_async_remote_copy(
                src_ref=qsA.at[slot], dst_ref=cA.at[s],
                send_sem=rsA_s.at[s], recv_sem=rsA_r.at[s],
                device_id=(right,), device_id_type=pl.DeviceIdType.MESH)
            rdmaB = pltpu.make_async_remote_copy(
                src_ref=qsB.at[slot], dst_ref=cB.at[s],
                send_sem=rsB_s.at[s], recv_sem=rsB_r.at[s],
                device_id=(left,), device_id_type=pl.DeviceIdType.MESH)
            scdA = pltpu.make_async_remote_copy(
                src_ref=scsA.at[pl.ds(slot, 1), :],
                dst_ref=scbA.at[pl.ds(s, 1), :],
                send_sem=scA_s.at[s], recv_sem=scA_r.at[s],
                device_id=(right,), device_id_type=pl.DeviceIdType.MESH)
            scdB = pltpu.make_async_remote_copy(
                src_ref=scsB.at[pl.ds(slot, 1), :],
                dst_ref=scbB.at[pl.ds(s, 1), :],
                send_sem=scB_s.at[s], recv_sem=scB_r.at[s],
                device_id=(left,), device_id_type=pl.DeviceIdType.MESH)
            rdmaA.start()
            rdmaB.start()
            scdA.start()
            scdB.start()
            pcA[...] = pchunk(mod(p - s - 1), 0)
            pcB[...] = pchunk(mod(p + s + 1), HN)
            rdmaA.wait()
            rdmaB.wait()
            scdA.wait()
            scdB.wait()
            accA[...] = cA[s].astype(jnp.float32) * scbA[s, 0] + pcA[...]
            accB[...] = cB[s].astype(jnp.float32) * scbB[s, 0] + pcB[...]
            if s < N_DEV - 2:
                q16(accA[...], qsA, scsA, (s + 1) % 2)
                q16(accB[...], qsB, scsB, (s + 1) % 2)

        ownA = mod(p + 1)
        ownB = mod(p - 1)

        my_amax = jnp.maximum(
            jnp.max(jnp.abs(accA[...])), jnp.max(jnp.abs(accB[...])))
        ax_src[0, :] = jnp.full((128,), my_amax, jnp.float32)
        ax_buf[pl.ds(me, 1), :] = jnp.full((1, 128), my_amax, jnp.float32)
        ax_rdmas = []
        for k in range(1, N_DEV):
            peer = mod(me + k)
            r = pltpu.make_async_remote_copy(
                src_ref=ax_src, dst_ref=ax_buf.at[pl.ds(me, 1), :],
                send_sem=ax_s.at[k - 1], recv_sem=ax_r.at[k - 1],
                device_id=(peer,), device_id_type=pl.DeviceIdType.MESH)
            r.start()
            ax_rdmas.append(r)
        for r in ax_rdmas:
            r.wait()
        amax = jnp.max(ax_buf[...])
        scale = amax / 127.0
        inv = 127.0 / amax

        qA[ownA] = jnp.clip(
            jnp.round(accA[...] * inv), -127.0, 127.0).astype(jnp.int8)
        qB[ownB] = jnp.clip(
            jnp.round(accB[...] * inv), -127.0, 127.0).astype(jnp.int8)
        pcA[...] = qA[ownA].astype(jnp.float32) * scale
        pcB[...] = qB[ownB].astype(jnp.float32) * scale
        pltpu.make_async_copy(
            pcA, out_ref.at[pl.ds(ownA * CH, CH), pl.ds(0, HN)],
            stA.at[2]).start()
        pltpu.make_async_copy(
            pcB, out_ref.at[pl.ds(ownB * CH, CH), pl.ds(HN, HN)],
            stB.at[2]).start()

        def rdA(t):
            g = mod(p + 1 - t)
            return pltpu.make_async_remote_copy(
                src_ref=qA.at[g], dst_ref=qA.at[g],
                send_sem=agA_s.at[t], recv_sem=agA_r.at[t],
                device_id=(right,), device_id_type=pl.DeviceIdType.MESH)

        def rdB(t):
            g = mod(p - 1 + t)
            return pltpu.make_async_remote_copy(
                src_ref=qB.at[g], dst_ref=qB.at[g],
                send_sem=agB_s.at[t], recv_sem=agB_r.at[t],
                device_id=(left,), device_id_type=pl.DeviceIdType.MESH)

        ra = rdA(0)
        rb = rdB(0)
        ra.start()
        rb.start()
        for t in range(N_DEV - 1):
            ra.wait()
            rb.wait()
            if t < N_DEV - 2:
                ra = rdA(t + 1)
                rb = rdB(t + 1)
                ra.start()
                rb.start()
            crA = mod(p - t)
            crB = mod(p + t)
            slot = t % 2
            if t >= 2:
                pltpu.make_async_copy(
                    dqA.at[slot], out_ref.at[pl.ds(0, CH), pl.ds(0, HN)],
                    stA.at[slot]).wait()
                pltpu.make_async_copy(
                    dqB.at[slot], out_ref.at[pl.ds(0, CH), pl.ds(HN, HN)],
                    stB.at[slot]).wait()
            dqA[slot] = qA[crA].astype(jnp.float32) * scale
            dqB[slot] = qB[crB].astype(jnp.float32) * scale
            pltpu.make_async_copy(
                dqA.at[slot], out_ref.at[pl.ds(crA * CH, CH), pl.ds(0, HN)],
                stA.at[slot]).start()
            pltpu.make_async_copy(
                dqB.at[slot], out_ref.at[pl.ds(crB * CH, CH), pl.ds(HN, HN)],
                stB.at[slot]).start()
        for sl in range(3):
            pltpu.make_async_copy(
                dqA.at[min(sl, 1)], out_ref.at[pl.ds(0, CH), pl.ds(0, HN)],
                stA.at[sl]).wait()
            pltpu.make_async_copy(
                dqB.at[min(sl, 1)], out_ref.at[pl.ds(0, CH), pl.ds(HN, HN)],
                stB.at[sl]).wait()

    nsl = N_DEV - 1
    return pl.pallas_call(
        body,
        out_shape=jax.ShapeDtypeStruct((M, N), jnp.float32),
        in_specs=[
            pl.BlockSpec(memory_space=pltpu.VMEM),
            pl.BlockSpec(memory_space=pltpu.VMEM),
        ],
        out_specs=pl.BlockSpec(memory_space=pl.ANY),
        scratch_shapes=[
            pltpu.VMEM((nsl, CH, HN), jnp.int16),
            pltpu.VMEM((nsl, CH, HN), jnp.int16),
            pltpu.VMEM((CH, HN), jnp.float32),
            pltpu.VMEM((CH, HN), jnp.float32),
            pltpu.VMEM((CH, HN), jnp.float32),
            pltpu.VMEM((CH, HN), jnp.float32),
            pltpu.VMEM((2, CH, HN), jnp.int16),
            pltpu.VMEM((2, CH, HN), jnp.int16),
            pltpu.VMEM((2, 128), jnp.float32),
            pltpu.VMEM((2, 128), jnp.float32),
            pltpu.VMEM((nsl, 128), jnp.float32),
            pltpu.VMEM((nsl, 128), jnp.float32),
            pltpu.VMEM((2, CH, HN), jnp.float32),
            pltpu.VMEM((2, CH, HN), jnp.float32),
            pltpu.VMEM((N_DEV, CH, HN), jnp.int8),
            pltpu.VMEM((N_DEV, CH, HN), jnp.int8),
            pltpu.VMEM((1, 128), jnp.float32),
            pltpu.VMEM((N_DEV, 128), jnp.float32),
            pltpu.VMEM((K, N), jnp.bfloat16),
            pltpu.VMEM((K, N), jnp.bfloat16),
            pltpu.SemaphoreType.DMA((nsl,)),
            pltpu.SemaphoreType.DMA((nsl,)),
            pltpu.SemaphoreType.DMA((nsl,)),
            pltpu.SemaphoreType.DMA((nsl,)),
            pltpu.SemaphoreType.DMA((nsl,)),
            pltpu.SemaphoreType.DMA((nsl,)),
            pltpu.SemaphoreType.DMA((nsl,)),
            pltpu.SemaphoreType.DMA((nsl,)),
            pltpu.SemaphoreType.DMA((nsl,)),
            pltpu.SemaphoreType.DMA((nsl,)),
            pltpu.SemaphoreType.DMA((nsl,)),
            pltpu.SemaphoreType.DMA((nsl,)),
            pltpu.SemaphoreType.DMA((nsl,)),
            pltpu.SemaphoreType.DMA((nsl,)),
            pltpu.SemaphoreType.DMA((3,)),
            pltpu.SemaphoreType.DMA((3,)),
        ],
        compiler_params=pltpu.CompilerParams(
            collective_id=0, vmem_limit_bytes=62 * 1024 * 1024),
    )(x, w_mat)


# device time: 191918 ns/iter; 4.2908x vs baseline; 1.0280x over previous
import jax
import jax.numpy as jnp
from jax import lax
from jax.experimental import pallas as pl
from jax.experimental.pallas import tpu as pltpu

N_DEV = 8
M = 4096
K = 512
N = 2048
HN = N // 2
CH = M // N_DEV


def mod(v):
    return lax.rem(v + 2 * N_DEV, N_DEV)


def perm(v):
    return jnp.where(v < 4, v, 11 - v)


def kernel(x, w_mat, quant=True):
    del quant

    def body(x_ref, w_ref, out_ref, cA, cB, pcA, pcB, accA, accB, qsA, qsB,
             scsA, scsB, scbA, scbB, dqA, dqB, qA, qB, ax_src, ax_buf,
             whi, wlo,
             rsA_s, rsA_r, rsB_s, rsB_r, scA_s, scA_r, scB_s, scB_r,
             agA_s, agA_r, agB_s, agB_r, ax_s, ax_r, stA, stB):
        me = lax.axis_index("i")
        p = perm(me)
        right = perm(mod(p + 1))
        left = perm(mod(p - 1))

        barrier = pltpu.get_barrier_semaphore()
        for nbr in (left, right):
            pl.semaphore_signal(
                barrier, inc=1, device_id=(nbr,),
                device_id_type=pl.DeviceIdType.MESH,
            )
        pl.semaphore_wait(barrier, 2)

        wf = w_ref[...]
        whi[...] = wf.astype(jnp.bfloat16)
        wlo[...] = (wf - whi[...].astype(jnp.float32)).astype(jnp.bfloat16)

        def bdot(a, b):
            return lax.dot_general(
                a, b, (((1,), (0,)), ((), ())),
                preferred_element_type=jnp.float32)

        def pchunk(c, col0):
            xf = x_ref[pl.ds(c * CH, CH), :]
            xhi = xf.astype(jnp.bfloat16)
            xlo = (xf - xhi.astype(jnp.float32)).astype(jnp.bfloat16)
            wh = whi[:, pl.ds(col0, HN)]
            return (bdot(xhi, wh)
                    + bdot(xhi, wlo[:, pl.ds(col0, HN)])
                    + bdot(xlo, wh))

        def q16(acc, qs_ref, scs_ref, slot):
            am = jnp.maximum(jnp.max(jnp.abs(acc)), 1e-30)
            qs_ref[slot] = jnp.clip(
                jnp.round(acc * (32767.0 / am)), -32767.0, 32767.0
            ).astype(jnp.int16)
            scs_ref[slot, :] = jnp.full((128,), am / 32767.0, jnp.float32)

        def rs_rdmas(d, s):
            slot = s % 2
            if d == 0:
                payload = pltpu.make_async_remote_copy(
                    src_ref=qsA.at[slot], dst_ref=cA.at[s],
                    send_sem=rsA_s.at[s], recv_sem=rsA_r.at[s],
                    device_id=(right,), device_id_type=pl.DeviceIdType.MESH)
                sc = pltpu.make_async_remote_copy(
                    src_ref=scsA.at[pl.ds(slot, 1), :],
                    dst_ref=scbA.at[pl.ds(s, 1), :],
                    send_sem=scA_s.at[s], recv_sem=scA_r.at[s],
                    device_id=(right,), device_id_type=pl.DeviceIdType.MESH)
            else:
                payload = pltpu.make_async_remote_copy(
                    src_ref=qsB.at[slot], dst_ref=cB.at[s],
                    send_sem=rsB_s.at[s], recv_sem=rsB_r.at[s],
                    device_id=(left,), device_id_type=pl.DeviceIdType.MESH)
                sc = pltpu.make_async_remote_copy(
                    src_ref=scsB.at[pl.ds(slot, 1), :],
                    dst_ref=scbB.at[pl.ds(s, 1), :],
                    send_sem=scB_s.at[s], recv_sem=scB_r.at[s],
                    device_id=(left,), device_id_type=pl.DeviceIdType.MESH)
            return payload, sc

        accA[...] = pchunk(p, 0)
        q16(accA[...], qsA, scsA, 0)
        rdmaA, scdA = rs_rdmas(0, 0)
        rdmaA.start()
        scdA.start()
        accB[...] = pchunk(p, HN)
        q16(accB[...], qsB, scsB, 0)
        rdmaB, scdB = rs_rdmas(1, 0)
        rdmaB.start()
        scdB.start()
        for s in range(N_DEV - 1):
            pcA[...] = pchunk(mod(p - s - 1), 0)
            pcB[...] = pchunk(mod(p + s + 1), HN)
            rdmaA.wait()
            scdA.wait()
            accA[...] = cA[s].astype(jnp.float32) * scbA[s, 0] + pcA[...]
            if s < N_DEV - 2:
                q16(accA[...], qsA, scsA, (s + 1) % 2)
                rdmaA, scdA = rs_rdmas(0, s + 1)
                rdmaA.start()
                scdA.start()
            rdmaB.wait()
            scdB.wait()
            accB[...] = cB[s].astype(jnp.float32) * scbB[s, 0] + pcB[...]
            if s < N_DEV - 2:
                q16(accB[...], qsB, scsB, (s + 1) % 2)
                rdmaB, scdB = rs_rdmas(1, s + 1)
                rdmaB.start()
                scdB.start()

        ownA = mod(p + 1)
        ownB = mod(p - 1)

        my_amax = jnp.maximum(
            jnp.max(jnp.abs(accA[...])), jnp.max(jnp.abs(accB[...])))
        ax_src[0, :] = jnp.full((128,), my_amax, jnp.float32)
        ax_buf[pl.ds(me, 1), :] = jnp.full((1, 128), my_amax, jnp.float32)
        ax_rdmas = []
        for k in range(1, N_DEV):
            peer = mod(me + k)
            r = pltpu.make_async_remote_copy(
                src_ref=ax_src, dst_ref=ax_buf.at[pl.ds(me, 1), :],
                send_sem=ax_s.at[k - 1], recv_sem=ax_r.at[k - 1],
                device_id=(peer,), device_id_type=pl.DeviceIdType.MESH)
            r.start()
            ax_rdmas.append(r)
        for r in ax_rdmas:
            r.wait()
        amax = jnp.max(ax_buf[...])
        scale = amax / 127.0
        inv = 127.0 / amax

        qA[ownA] = jnp.clip(
            jnp.round(accA[...] * inv), -127.0, 127.0).astype(jnp.int8)
        qB[ownB] = jnp.clip(
            jnp.round(accB[...] * inv), -127.0, 127.0).astype(jnp.int8)
        pcA[...] = qA[ownA].astype(jnp.float32) * scale
        pcB[...] = qB[ownB].astype(jnp.float32) * scale
        pltpu.make_async_copy(
            pcA, out_ref.at[pl.ds(ownA * CH, CH), pl.ds(0, HN)],
            stA.at[2]).start()
        pltpu.make_async_copy(
            pcB, out_ref.at[pl.ds(ownB * CH, CH), pl.ds(HN, HN)],
            stB.at[2]).start()

        def rdA(t):
            g = mod(p + 1 - t)
            return pltpu.make_async_remote_copy(
                src_ref=qA.at[g], dst_ref=qA.at[g],
                send_sem=agA_s.at[t], recv_sem=agA_r.at[t],
                device_id=(right,), device_id_type=pl.DeviceIdType.MESH)

        def rdB(t):
            g = mod(p - 1 + t)
            return pltpu.make_async_remote_copy(
                src_ref=qB.at[g], dst_ref=qB.at[g],
                send_sem=agB_s.at[t], recv_sem=agB_r.at[t],
                device_id=(left,), device_id_type=pl.DeviceIdType.MESH)

        ra = rdA(0)
        rb = rdB(0)
        ra.start()
        rb.start()
        for t in range(N_DEV - 1):
            ra.wait()
            if t < N_DEV - 2:
                ra = rdA(t + 1)
                ra.start()
            rb.wait()
            if t < N_DEV - 2:
                rb = rdB(t + 1)
                rb.start()
            crA = mod(p - t)
            crB = mod(p + t)
            slot = t % 2
            if t >= 2:
                pltpu.make_async_copy(
                    dqA.at[slot], out_ref.at[pl.ds(0, CH), pl.ds(0, HN)],
                    stA.at[slot]).wait()
                pltpu.make_async_copy(
                    dqB.at[slot], out_ref.at[pl.ds(0, CH), pl.ds(HN, HN)],
                    stB.at[slot]).wait()
            dqA[slot] = qA[crA].astype(jnp.float32) * scale
            dqB[slot] = qB[crB].astype(jnp.float32) * scale
            pltpu.make_async_copy(
                dqA.at[slot], out_ref.at[pl.ds(crA * CH, CH), pl.ds(0, HN)],
                stA.at[slot]).start()
            pltpu.make_async_copy(
                dqB.at[slot], out_ref.at[pl.ds(crB * CH, CH), pl.ds(HN, HN)],
                stB.at[slot]).start()
        for sl in range(3):
            pltpu.make_async_copy(
                dqA.at[min(sl, 1)], out_ref.at[pl.ds(0, CH), pl.ds(0, HN)],
                stA.at[sl]).wait()
            pltpu.make_async_copy(
                dqB.at[min(sl, 1)], out_ref.at[pl.ds(0, CH), pl.ds(HN, HN)],
                stB.at[sl]).wait()

    nsl = N_DEV - 1
    return pl.pallas_call(
        body,
        out_shape=jax.ShapeDtypeStruct((M, N), jnp.float32),
        in_specs=[
            pl.BlockSpec(memory_space=pltpu.VMEM),
            pl.BlockSpec(memory_space=pltpu.VMEM),
        ],
        out_specs=pl.BlockSpec(memory_space=pl.ANY),
        scratch_shapes=[
            pltpu.VMEM((nsl, CH, HN), jnp.int16),
            pltpu.VMEM((nsl, CH, HN), jnp.int16),
            pltpu.VMEM((CH, HN), jnp.float32),
            pltpu.VMEM((CH, HN), jnp.float32),
            pltpu.VMEM((CH, HN), jnp.float32),
            pltpu.VMEM((CH, HN), jnp.float32),
            pltpu.VMEM((2, CH, HN), jnp.int16),
            pltpu.VMEM((2, CH, HN), jnp.int16),
            pltpu.VMEM((2, 128), jnp.float32),
            pltpu.VMEM((2, 128), jnp.float32),
            pltpu.VMEM((nsl, 128), jnp.float32),
            pltpu.VMEM((nsl, 128), jnp.float32),
            pltpu.VMEM((2, CH, HN), jnp.float32),
            pltpu.VMEM((2, CH, HN), jnp.float32),
            pltpu.VMEM((N_DEV, CH, HN), jnp.int8),
            pltpu.VMEM((N_DEV, CH, HN), jnp.int8),
            pltpu.VMEM((1, 128), jnp.float32),
            pltpu.VMEM((N_DEV, 128), jnp.float32),
            pltpu.VMEM((K, N), jnp.bfloat16),
            pltpu.VMEM((K, N), jnp.bfloat16),
            pltpu.SemaphoreType.DMA((nsl,)),
            pltpu.SemaphoreType.DMA((nsl,)),
            pltpu.SemaphoreType.DMA((nsl,)),
            pltpu.SemaphoreType.DMA((nsl,)),
            pltpu.SemaphoreType.DMA((nsl,)),
            pltpu.SemaphoreType.DMA((nsl,)),
            pltpu.SemaphoreType.DMA((nsl,)),
            pltpu.SemaphoreType.DMA((nsl,)),
            pltpu.SemaphoreType.DMA((nsl,)),
            pltpu.SemaphoreType.DMA((nsl,)),
            pltpu.SemaphoreType.DMA((nsl,)),
            pltpu.SemaphoreType.DMA((nsl,)),
            pltpu.SemaphoreType.DMA((nsl,)),
            pltpu.SemaphoreType.DMA((nsl,)),
            pltpu.SemaphoreType.DMA((3,)),
            pltpu.SemaphoreType.DMA((3,)),
        ],
        compiler_params=pltpu.CompilerParams(
            collective_id=0, vmem_limit_bytes=62 * 1024 * 1024),
    )(x, w_mat)
